# baseline (speedup 1.0000x reference)
import jax
import jax.numpy as jnp
from jax.experimental import pallas as pl
from jax.experimental.pallas import tpu as pltpu

_DFF = 4096
_DMODEL = 1024
_K = 1024
_MB = 512
_NBLK = _DFF // _MB
_BB = 256
_NBB = _DFF // _BB
_XROWS = 2 * 4096
_NSTEP = 32
_XB = _XROWS // _NSTEP


def _bottom_k_mask(m):
    u = jax.lax.bitcast_convert_type(m, jnp.int32)
    k = jnp.int32(_K)

    def bs_body(_, carry):
        lo, hi = carry
        mid = lo + (hi - lo) // 2
        c = jnp.sum((u <= mid).astype(jnp.int32))
        take = c >= k
        return jnp.where(take, lo, mid + 1), jnp.where(take, mid, hi)

    _, p = jax.lax.fori_loop(
        0, 31, bs_body, (jnp.int32(0), jnp.int32(0x7F800000)))

    lt = u < p
    eq = u == p
    c_lt = jnp.sum(lt.astype(jnp.int32))
    need = k - c_lt

    e = eq.astype(jnp.int32)
    x = e
    s = 1
    while s < _MB:
        sh = jnp.concatenate([jnp.zeros((_NBLK, s), jnp.int32), x[:, :-s]],
                             axis=1)
        x = x + sh
        s *= 2
    row_tot = x[:, _MB - 1:_MB]
    y = row_tot
    s = 1
    while s < _NBLK:
        shy = jnp.concatenate([jnp.zeros((s, 1), jnp.int32), y[:-s, :]],
                              axis=0)
        y = y + shy
        s *= 2
    row_off = jnp.concatenate([jnp.zeros((1, 1), jnp.int32), y[:-1, :]],
                              axis=0)
    excl = (x - e) + row_off
    prune_eq = eq & (excl < need)
    keep = jnp.logical_not(jnp.logical_or(lt, prune_eq))
    return keep.astype(jnp.float32)


def _fused_body(w1_ref, w2_ref, f1_ref, f2_ref,
                maskout_ref, w1out_ref, w2out_ref,
                mags_s, mask_s):
    i = pl.program_id(0)

    @pl.when(i < _NBLK)
    def _mags_phase():
        w1 = w1_ref[pl.ds(i * _MB, _MB), :]
        w2 = w2_ref[:, pl.ds(i * _MB, _MB)]
        s1 = jnp.sum(w1 * w1, axis=1)
        s2 = jnp.sum(w2 * w2, axis=0)
        mags_s[pl.ds(i, 1), :] = (jnp.sqrt(s1) * jnp.sqrt(s2)).reshape(1, _MB)

    @pl.when(i == _NBLK)
    def _mask_phase():
        mask = _bottom_k_mask(mags_s[...])
        mask_s[...] = mask
        maskout_ref[...] = mask

    @pl.when(jnp.logical_and(i >= _NBLK, i < _NBLK + _NBB))
    def _blend_phase():
        j = i - _NBLK
        mrow = mask_s[pl.ds(j // 2, 1), pl.ds((j % 2) * _BB, _BB)]
        keep_r = mrow > 0.5
        w2blk = w2_ref[:, pl.ds(j * _BB, _BB)]
        w2out_ref[...] = jnp.where(keep_r, w2blk, f2_ref[...])

        ii = jax.lax.broadcasted_iota(jnp.int32, (_BB, _BB), 0)
        jj = jax.lax.broadcasted_iota(jnp.int32, (_BB, _BB), 1)
        m_b = jnp.broadcast_to(mrow, (_BB, _BB))
        mcol = jnp.sum(jnp.where(ii == jj, m_b, 0.0), axis=1, keepdims=True)
        keep_c = mcol > 0.5
        w1blk = w1_ref[pl.ds(j * _BB, _BB), :]
        w1out_ref[...] = jnp.where(keep_c, w1blk, f1_ref[...])


_NB = 4
_CHR = 256
_NCH = _XROWS // _CHR


def _relu_ring_body(x_hbm, y_hbm, ibuf, obuf, isem, osem):
    def in_copy(c, b):
        return pltpu.make_async_copy(
            x_hbm.at[pl.ds(c * _CHR, _CHR), :], ibuf.at[b], isem.at[b])

    def out_copy(c, b):
        return pltpu.make_async_copy(
            obuf.at[b], y_hbm.at[pl.ds(c * _CHR, _CHR), :], osem.at[b])

    for b in range(_NB):
        in_copy(b, b).start()

    def step(c, _):
        b = jax.lax.rem(c, _NB)

        @pl.when(c >= _NB)
        def _wait_prev_out():
            out_copy(c - _NB, b).wait()

        in_copy(c, b).wait()
        obuf[b] = jnp.maximum(ibuf[b], 0.0)
        out_copy(c, b).start()

        @pl.when(c + _NB < _NCH)
        def _next_in():
            in_copy(c + _NB, b).start()

        return 0

    jax.lax.fori_loop(0, _NCH, step, 0)
    for c in range(_NCH - _NB, _NCH):
        out_copy(c, c % _NB).wait()


def _relu_ring(x2):
    return pl.pallas_call(
        _relu_ring_body,
        in_specs=[pl.BlockSpec(memory_space=pl.ANY)],
        out_specs=pl.BlockSpec(memory_space=pl.ANY),
        out_shape=jax.ShapeDtypeStruct((_XROWS, _DMODEL), jnp.float32),
        scratch_shapes=[
            pltpu.VMEM((_NB, _CHR, _DMODEL), jnp.float32),
            pltpu.VMEM((_NB, _CHR, _DMODEL), jnp.float32),
            pltpu.SemaphoreType.DMA((_NB,)),
            pltpu.SemaphoreType.DMA((_NB,)),
        ],
    )(x2)


def kernel(x, W1, W2, frozen1, frozen2, target1, target2):
    del target1, target2

    x2 = x.reshape(_XROWS, _DMODEL)

    def _bmap(i):
        return jnp.clip(i - _NBLK, 0, _NBB - 1)

    mask2d, W1_new, W2_new = pl.pallas_call(
        _fused_body,
        grid=(_NSTEP,),
        in_specs=[
            pl.BlockSpec((_DFF, _DMODEL), lambda i: (0, 0)),
            pl.BlockSpec((_DMODEL, _DFF), lambda i: (0, 0)),
            pl.BlockSpec((_BB, _DMODEL), lambda i: (_bmap(i), 0)),
            pl.BlockSpec((_DMODEL, _BB), lambda i: (0, _bmap(i))),
        ],
        out_specs=[
            pl.BlockSpec((_NBLK, _MB), lambda i: (0, 0)),
            pl.BlockSpec((_BB, _DMODEL), lambda i: (_bmap(i), 0)),
            pl.BlockSpec((_DMODEL, _BB), lambda i: (0, _bmap(i))),
        ],
        out_shape=[
            jax.ShapeDtypeStruct((_NBLK, _MB), jnp.float32),
            jax.ShapeDtypeStruct((_DFF, _DMODEL), jnp.float32),
            jax.ShapeDtypeStruct((_DMODEL, _DFF), jnp.float32),
        ],
        scratch_shapes=[
            pltpu.VMEM((_NBLK, _MB), jnp.float32),
            pltpu.VMEM((_NBLK, _MB), jnp.float32),
        ],
    )(W1, W2, frozen1, frozen2)

    mask = mask2d.reshape(_DFF)
    y = _relu_ring(x2)
    return y.reshape(x.shape), W1_new, W2_new, mask


# speedup vs baseline: 1.0442x; 1.0442x over previous
import jax
import jax.numpy as jnp
from jax.experimental import pallas as pl
from jax.experimental.pallas import tpu as pltpu

_DFF = 4096
_DMODEL = 1024
_K = 1024
_MB = 512
_NBLK = _DFF // _MB
_XROWS = 2 * 4096


def _bottom_k_mask(m):
    u = jax.lax.bitcast_convert_type(m, jnp.int32)
    k = jnp.int32(_K)

    def bs_body(_, carry):
        lo, hi = carry
        mid = lo + (hi - lo) // 2
        c = jnp.sum((u <= mid).astype(jnp.int32))
        take = c >= k
        return jnp.where(take, lo, mid + 1), jnp.where(take, mid, hi)

    _, p = jax.lax.fori_loop(
        0, 31, bs_body, (jnp.int32(0), jnp.int32(0x7F800000)))

    lt = u < p
    eq = u == p
    c_lt = jnp.sum(lt.astype(jnp.int32))
    need = k - c_lt

    e = eq.astype(jnp.int32)
    x = e
    s = 1
    while s < _MB:
        sh = jnp.concatenate([jnp.zeros((_NBLK, s), jnp.int32), x[:, :-s]],
                             axis=1)
        x = x + sh
        s *= 2
    row_tot = x[:, _MB - 1:_MB]
    y = row_tot
    s = 1
    while s < _NBLK:
        shy = jnp.concatenate([jnp.zeros((s, 1), jnp.int32), y[:-s, :]],
                              axis=0)
        y = y + shy
        s *= 2
    row_off = jnp.concatenate([jnp.zeros((1, 1), jnp.int32), y[:-1, :]],
                              axis=0)
    excl = (x - e) + row_off
    prune_eq = eq & (excl < need)
    keep = jnp.logical_not(jnp.logical_or(lt, prune_eq))
    return keep.astype(jnp.float32)


def _fused_body(w1_ref, w2_ref, f1_ref, f2_ref,
                maskout_ref, w1out_ref, w2out_ref,
                mags_s, mask_s):
    i = pl.program_id(0)

    @pl.when(i < _NBLK)
    def _mags_phase():
        w1 = w1_ref[pl.ds(i * _MB, _MB), :]
        w2 = w2_ref[:, pl.ds(i * _MB, _MB)]
        s1 = jnp.sum(w1 * w1, axis=1)
        s2 = jnp.sum(w2 * w2, axis=0)
        mags_s[pl.ds(i, 1), :] = (jnp.sqrt(s1) * jnp.sqrt(s2)).reshape(1, _MB)

    @pl.when(i == _NBLK)
    def _mask_phase():
        mask = _bottom_k_mask(mags_s[...])
        mask_s[...] = mask
        maskout_ref[...] = mask

    @pl.when(i >= _NBLK)
    def _blend_phase():
        j = i - _NBLK
        mrow = mask_s[pl.ds(j, 1), :]
        keep_r = mrow > 0.5
        w2blk = w2_ref[:, pl.ds(j * _MB, _MB)]
        w2out_ref[...] = jnp.where(keep_r, w2blk, f2_ref[...])

        ii = jax.lax.broadcasted_iota(jnp.int32, (_MB, _MB), 0)
        jj = jax.lax.broadcasted_iota(jnp.int32, (_MB, _MB), 1)
        m_b = jnp.broadcast_to(mrow, (_MB, _MB))
        mcol = jnp.sum(jnp.where(ii == jj, m_b, 0.0), axis=1, keepdims=True)
        keep_c = mcol > 0.5
        w1blk = w1_ref[pl.ds(j * _MB, _MB), :]
        w1out_ref[...] = jnp.where(keep_c, w1blk, f1_ref[...])


_NB = 4
_CHR = 256
_NCH = _XROWS // _CHR


def _relu_ring_body(x_hbm, y_hbm, ibuf, obuf, isem, osem):
    def in_copy(c, b):
        return pltpu.make_async_copy(
            x_hbm.at[pl.ds(c * _CHR, _CHR), :], ibuf.at[b], isem.at[b])

    def out_copy(c, b):
        return pltpu.make_async_copy(
            obuf.at[b], y_hbm.at[pl.ds(c * _CHR, _CHR), :], osem.at[b])

    for b in range(_NB):
        in_copy(b, b).start()

    def step(c, _):
        b = jax.lax.rem(c, _NB)

        @pl.when(c >= _NB)
        def _wait_prev_out():
            out_copy(c - _NB, b).wait()

        in_copy(c, b).wait()
        obuf[b] = jnp.maximum(ibuf[b], 0.0)
        out_copy(c, b).start()

        @pl.when(c + _NB < _NCH)
        def _next_in():
            in_copy(c + _NB, b).start()

        return 0

    jax.lax.fori_loop(0, _NCH, step, 0)
    for c in range(_NCH - _NB, _NCH):
        out_copy(c, c % _NB).wait()


def _relu_ring(x2):
    return pl.pallas_call(
        _relu_ring_body,
        in_specs=[pl.BlockSpec(memory_space=pl.ANY)],
        out_specs=pl.BlockSpec(memory_space=pl.ANY),
        out_shape=jax.ShapeDtypeStruct((_XROWS, _DMODEL), jnp.float32),
        scratch_shapes=[
            pltpu.VMEM((_NB, _CHR, _DMODEL), jnp.float32),
            pltpu.VMEM((_NB, _CHR, _DMODEL), jnp.float32),
            pltpu.SemaphoreType.DMA((_NB,)),
            pltpu.SemaphoreType.DMA((_NB,)),
        ],
    )(x2)


def kernel(x, W1, W2, frozen1, frozen2, target1, target2):
    del target1, target2

    x2 = x.reshape(_XROWS, _DMODEL)

    def _bmap(i):
        return jnp.maximum(i - _NBLK, 0)

    mask2d, W1_new, W2_new = pl.pallas_call(
        _fused_body,
        grid=(2 * _NBLK,),
        in_specs=[
            pl.BlockSpec((_DFF, _DMODEL), lambda i: (0, 0)),
            pl.BlockSpec((_DMODEL, _DFF), lambda i: (0, 0)),
            pl.BlockSpec((_MB, _DMODEL), lambda i: (_bmap(i), 0)),
            pl.BlockSpec((_DMODEL, _MB), lambda i: (0, _bmap(i))),
        ],
        out_specs=[
            pl.BlockSpec((_NBLK, _MB), lambda i: (0, 0)),
            pl.BlockSpec((_MB, _DMODEL), lambda i: (_bmap(i), 0)),
            pl.BlockSpec((_DMODEL, _MB), lambda i: (0, _bmap(i))),
        ],
        out_shape=[
            jax.ShapeDtypeStruct((_NBLK, _MB), jnp.float32),
            jax.ShapeDtypeStruct((_DFF, _DMODEL), jnp.float32),
            jax.ShapeDtypeStruct((_DMODEL, _DFF), jnp.float32),
        ],
        scratch_shapes=[
            pltpu.VMEM((_NBLK, _MB), jnp.float32),
            pltpu.VMEM((_NBLK, _MB), jnp.float32),
        ],
    )(W1, W2, frozen1, frozen2)

    mask = mask2d.reshape(_DFF)
    y = _relu_ring(x2)
    return y.reshape(x.shape), W1_new, W2_new, mask


# speedup vs baseline: 1.1144x; 1.0672x over previous
import jax
import jax.numpy as jnp
from jax.experimental import pallas as pl
from jax.experimental.pallas import tpu as pltpu

_DFF = 4096
_DMODEL = 1024
_K = 1024
_MB = 512
_NBLK = _DFF // _MB
_XROWS = 2 * 4096


def _bottom_k_mask(m):
    u = jax.lax.bitcast_convert_type(m, jnp.int32)
    k = jnp.int32(_K)

    def bs_body(_, carry):
        lo, hi = carry
        mid = lo + (hi - lo) // 2
        c = jnp.sum((u <= mid).astype(jnp.int32))
        take = c >= k
        return jnp.where(take, lo, mid + 1), jnp.where(take, mid, hi)

    _, p = jax.lax.fori_loop(
        0, 31, bs_body, (jnp.int32(0), jnp.int32(0x7F800000)))

    lt = u < p
    eq = u == p
    c_lt = jnp.sum(lt.astype(jnp.int32))
    need = k - c_lt

    e = eq.astype(jnp.int32)
    x = e
    s = 1
    while s < _MB:
        sh = jnp.concatenate([jnp.zeros((_NBLK, s), jnp.int32), x[:, :-s]],
                             axis=1)
        x = x + sh
        s *= 2
    row_tot = x[:, _MB - 1:_MB]
    y = row_tot
    s = 1
    while s < _NBLK:
        shy = jnp.concatenate([jnp.zeros((s, 1), jnp.int32), y[:-s, :]],
                              axis=0)
        y = y + shy
        s *= 2
    row_off = jnp.concatenate([jnp.zeros((1, 1), jnp.int32), y[:-1, :]],
                              axis=0)
    excl = (x - e) + row_off
    prune_eq = eq & (excl < need)
    keep = jnp.logical_not(jnp.logical_or(lt, prune_eq))
    return keep.astype(jnp.float32)


_W2LC = 8
_W2LR = _DMODEL // _W2LC

_B1 = 256
_N1 = _DFF // _B1
_B2 = 64
_N2 = _DMODEL // _B2
_NB = 4


def _bottom_k_mask(m):
    u = jax.lax.bitcast_convert_type(m, jnp.int32)
    k = jnp.int32(_K)

    def bs_body(_, carry):
        lo, hi = carry
        mid = lo + (hi - lo) // 2
        c = jnp.sum((u <= mid).astype(jnp.int32))
        take = c >= k
        return jnp.where(take, lo, mid + 1), jnp.where(take, mid, hi)

    _, p = jax.lax.fori_loop(
        0, 31, bs_body, (jnp.int32(0), jnp.int32(0x7F800000)))

    lt = u < p
    eq = u == p
    c_lt = jnp.sum(lt.astype(jnp.int32))
    need = k - c_lt

    e = eq.astype(jnp.int32)
    x = e
    s = 1
    while s < _MB:
        sh = jnp.concatenate([jnp.zeros((_NBLK, s), jnp.int32), x[:, :-s]],
                             axis=1)
        x = x + sh
        s *= 2
    row_tot = x[:, _MB - 1:_MB]
    y = row_tot
    s = 1
    while s < _NBLK:
        shy = jnp.concatenate([jnp.zeros((s, 1), jnp.int32), y[:-s, :]],
                              axis=0)
        y = y + shy
        s *= 2
    row_off = jnp.concatenate([jnp.zeros((1, 1), jnp.int32), y[:-1, :]],
                              axis=0)
    excl = (x - e) + row_off
    prune_eq = eq & (excl < need)
    keep = jnp.logical_not(jnp.logical_or(lt, prune_eq))
    return keep.astype(jnp.float32)


def _fused_manual_body(w1_hbm, w2_hbm, f1_hbm, f2_hbm,
                       maskout_hbm, w1o_hbm, w2o_hbm,
                       w1v, w2v, s1_s, mags_s, mask_s, maskrow_s,
                       f1b, w1ob, f2b, w2ob,
                       sw1, sw2, sf1i, sf1o, sf2i, sf2o, smask):
    def w1_load(i):
        return pltpu.make_async_copy(
            w1_hbm.at[pl.ds(i * _MB, _MB), :],
            w1v.at[pl.ds(i * _MB, _MB), :], sw1.at[i])

    def w2_load(i):
        return pltpu.make_async_copy(
            w2_hbm.at[pl.ds(i * _W2LR, _W2LR), :],
            w2v.at[pl.ds(i * _W2LR, _W2LR), :], sw2.at[i])

    def f1_in(c, b):
        return pltpu.make_async_copy(
            f1_hbm.at[pl.ds(c * _B1, _B1), :], f1b.at[b], sf1i.at[b])

    def w1_out(c, b):
        return pltpu.make_async_copy(
            w1ob.at[b], w1o_hbm.at[pl.ds(c * _B1, _B1), :], sf1o.at[b])

    def f2_in(c, b):
        return pltpu.make_async_copy(
            f2_hbm.at[pl.ds(c * _B2, _B2), :], f2b.at[b], sf2i.at[b])

    def w2_out(c, b):
        return pltpu.make_async_copy(
            w2ob.at[b], w2o_hbm.at[pl.ds(c * _B2, _B2), :], sf2o.at[b])

    for i in range(_NBLK):
        w1_load(i).start()
    for i in range(_W2LC):
        w2_load(i).start()

    def mags1_step(i, _):
        w1_load(i).wait()
        w1 = w1v[pl.ds(i * _MB, _MB), :]
        s1 = jnp.sum(w1 * w1, axis=1)
        s1_s[pl.ds(i, 1), :] = jnp.sqrt(s1).reshape(1, _MB)
        return 0

    jax.lax.fori_loop(0, _NBLK, mags1_step, 0)

    for i in range(_W2LC):
        w2_load(i).wait()

    for b in range(_NB):
        f1_in(b, b).start()
        f2_in(b, b).start()

    def mags2_step(i, _):
        w2 = w2v[:, pl.ds(i * _MB, _MB)]
        s2 = jnp.sum(w2 * w2, axis=0)
        mags_s[pl.ds(i, 1), :] = (
            s1_s[pl.ds(i, 1), :] * jnp.sqrt(s2).reshape(1, _MB))
        return 0

    jax.lax.fori_loop(0, _NBLK, mags2_step, 0)

    mask = _bottom_k_mask(mags_s[...])
    mask_s[...] = mask
    pltpu.make_async_copy(mask_s, maskout_hbm, smask).start()
    for i in range(_NBLK):
        maskrow_s[0:1, pl.ds(i * _MB, _MB)] = mask_s[pl.ds(i, 1), :]

    def blend1_step(c, _):
        b = jax.lax.rem(c, _NB)

        @pl.when(c >= _NB)
        def _wait_out():
            w1_out(c - _NB, b).wait()

        f1_in(c, b).wait()
        mrow = maskrow_s[0:1, pl.ds(c * _B1, _B1)]
        ii = jax.lax.broadcasted_iota(jnp.int32, (_B1, _B1), 0)
        jj = jax.lax.broadcasted_iota(jnp.int32, (_B1, _B1), 1)
        m_b = jnp.broadcast_to(mrow, (_B1, _B1))
        mcol = jnp.sum(jnp.where(ii == jj, m_b, 0.0), axis=1, keepdims=True)
        w1ob[b] = jnp.where(mcol > 0.5, w1v[pl.ds(c * _B1, _B1), :], f1b[b])
        w1_out(c, b).start()

        @pl.when(c + _NB < _N1)
        def _next_in():
            f1_in(c + _NB, b).start()

        return 0

    jax.lax.fori_loop(0, _N1, blend1_step, 0)

    def blend2_step(c, _):
        b = jax.lax.rem(c, _NB)

        @pl.when(c >= _NB)
        def _wait_out():
            w2_out(c - _NB, b).wait()

        f2_in(c, b).wait()
        keep = maskrow_s[...] > 0.5
        w2ob[b] = jnp.where(keep, w2v[pl.ds(c * _B2, _B2), :], f2b[b])
        w2_out(c, b).start()

        @pl.when(c + _NB < _N2)
        def _next_in():
            f2_in(c + _NB, b).start()

        return 0

    jax.lax.fori_loop(0, _N2, blend2_step, 0)

    for c in range(_N1 - _NB, _N1):
        w1_out(c, c % _NB).wait()
    for c in range(_N2 - _NB, _N2):
        w2_out(c, c % _NB).wait()
    pltpu.make_async_copy(mask_s, maskout_hbm, smask).wait()


def _fused_manual(W1, W2, frozen1, frozen2):
    return pl.pallas_call(
        _fused_manual_body,
        in_specs=[pl.BlockSpec(memory_space=pl.ANY)] * 4,
        out_specs=[pl.BlockSpec(memory_space=pl.ANY)] * 3,
        out_shape=[
            jax.ShapeDtypeStruct((_NBLK, _MB), jnp.float32),
            jax.ShapeDtypeStruct((_DFF, _DMODEL), jnp.float32),
            jax.ShapeDtypeStruct((_DMODEL, _DFF), jnp.float32),
        ],
        scratch_shapes=[
            pltpu.VMEM((_DFF, _DMODEL), jnp.float32),
            pltpu.VMEM((_DMODEL, _DFF), jnp.float32),
            pltpu.VMEM((_NBLK, _MB), jnp.float32),
            pltpu.VMEM((_NBLK, _MB), jnp.float32),
            pltpu.VMEM((_NBLK, _MB), jnp.float32),
            pltpu.VMEM((1, _DFF), jnp.float32),
            pltpu.VMEM((_NB, _B1, _DMODEL), jnp.float32),
            pltpu.VMEM((_NB, _B1, _DMODEL), jnp.float32),
            pltpu.VMEM((_NB, _B2, _DFF), jnp.float32),
            pltpu.VMEM((_NB, _B2, _DFF), jnp.float32),
            pltpu.SemaphoreType.DMA((_NBLK,)),
            pltpu.SemaphoreType.DMA((_W2LC,)),
            pltpu.SemaphoreType.DMA((_NB,)),
            pltpu.SemaphoreType.DMA((_NB,)),
            pltpu.SemaphoreType.DMA((_NB,)),
            pltpu.SemaphoreType.DMA((_NB,)),
            pltpu.SemaphoreType.DMA,
        ],
    )(W1, W2, frozen1, frozen2)


_NB = 4
_CHR = 256
_NCH = _XROWS // _CHR


def _relu_ring_body(x_hbm, y_hbm, ibuf, obuf, isem, osem):
    def in_copy(c, b):
        return pltpu.make_async_copy(
            x_hbm.at[pl.ds(c * _CHR, _CHR), :], ibuf.at[b], isem.at[b])

    def out_copy(c, b):
        return pltpu.make_async_copy(
            obuf.at[b], y_hbm.at[pl.ds(c * _CHR, _CHR), :], osem.at[b])

    for b in range(_NB):
        in_copy(b, b).start()

    def step(c, _):
        b = jax.lax.rem(c, _NB)

        @pl.when(c >= _NB)
        def _wait_prev_out():
            out_copy(c - _NB, b).wait()

        in_copy(c, b).wait()
        obuf[b] = jnp.maximum(ibuf[b], 0.0)
        out_copy(c, b).start()

        @pl.when(c + _NB < _NCH)
        def _next_in():
            in_copy(c + _NB, b).start()

        return 0

    jax.lax.fori_loop(0, _NCH, step, 0)
    for c in range(_NCH - _NB, _NCH):
        out_copy(c, c % _NB).wait()


def _relu_ring(x2):
    return pl.pallas_call(
        _relu_ring_body,
        in_specs=[pl.BlockSpec(memory_space=pl.ANY)],
        out_specs=pl.BlockSpec(memory_space=pl.ANY),
        out_shape=jax.ShapeDtypeStruct((_XROWS, _DMODEL), jnp.float32),
        scratch_shapes=[
            pltpu.VMEM((_NB, _CHR, _DMODEL), jnp.float32),
            pltpu.VMEM((_NB, _CHR, _DMODEL), jnp.float32),
            pltpu.SemaphoreType.DMA((_NB,)),
            pltpu.SemaphoreType.DMA((_NB,)),
        ],
    )(x2)


def kernel(x, W1, W2, frozen1, frozen2, target1, target2):
    del target1, target2

    x2 = x.reshape(_XROWS, _DMODEL)

    mask2d, W1_new, W2_new = _fused_manual(W1, W2, frozen1, frozen2)

    mask = mask2d.reshape(_DFF)
    y = _relu_ring(x2)
    return y.reshape(x.shape), W1_new, W2_new, mask


# speedup vs baseline: 1.2017x; 1.0783x over previous
import jax
import jax.numpy as jnp
from jax.experimental import pallas as pl
from jax.experimental.pallas import tpu as pltpu

_DFF = 4096
_DMODEL = 1024
_K = 1024
_MB = 512
_NBLK = _DFF // _MB
_XROWS = 2 * 4096


_W2LC = 8
_W2LR = _DMODEL // _W2LC

_B1 = 256
_N1 = _DFF // _B1
_B2 = 64
_N2 = _DMODEL // _B2
_NB = 4
_RB = 128
_NR = _XROWS // _RB
_RNB = 8


def _bottom_k_mask(m):
    u = jax.lax.bitcast_convert_type(m, jnp.int32)
    k = jnp.int32(_K)

    def bs_body(_, carry):
        lo, hi = carry
        mid = lo + (hi - lo) // 2
        c = jnp.sum((u <= mid).astype(jnp.int32))
        take = c >= k
        return jnp.where(take, lo, mid + 1), jnp.where(take, mid, hi)

    _, p = jax.lax.fori_loop(
        0, 31, bs_body, (jnp.int32(0), jnp.int32(0x7F800000)))

    lt = u < p
    eq = u == p
    c_lt = jnp.sum(lt.astype(jnp.int32))
    need = k - c_lt

    e = eq.astype(jnp.int32)
    x = e
    s = 1
    while s < _MB:
        sh = jnp.concatenate([jnp.zeros((_NBLK, s), jnp.int32), x[:, :-s]],
                             axis=1)
        x = x + sh
        s *= 2
    row_tot = x[:, _MB - 1:_MB]
    y = row_tot
    s = 1
    while s < _NBLK:
        shy = jnp.concatenate([jnp.zeros((s, 1), jnp.int32), y[:-s, :]],
                              axis=0)
        y = y + shy
        s *= 2
    row_off = jnp.concatenate([jnp.zeros((1, 1), jnp.int32), y[:-1, :]],
                              axis=0)
    excl = (x - e) + row_off
    prune_eq = eq & (excl < need)
    keep = jnp.logical_not(jnp.logical_or(lt, prune_eq))
    return keep.astype(jnp.float32)


def _fused_manual_body(w1_hbm, w2_hbm, f1_hbm, f2_hbm, x_hbm,
                       maskout_hbm, w1o_hbm, w2o_hbm, y_hbm,
                       w1v, w2v, s1_s, mags_s, mask_s, maskrow_s,
                       f1b, w1ob, f2b, w2ob, rib, rob,
                       sw1, sw2, sf1i, sf1o, sf2i, sf2o, sri, sro, smask):
    def w1_load(i):
        return pltpu.make_async_copy(
            w1_hbm.at[pl.ds(i * _MB, _MB), :],
            w1v.at[pl.ds(i * _MB, _MB), :], sw1.at[i])

    def w2_load(i):
        return pltpu.make_async_copy(
            w2_hbm.at[pl.ds(i * _W2LR, _W2LR), :],
            w2v.at[pl.ds(i * _W2LR, _W2LR), :], sw2.at[i])

    def f1_in(c, b):
        return pltpu.make_async_copy(
            f1_hbm.at[pl.ds(c * _B1, _B1), :], f1b.at[b], sf1i.at[b])

    def w1_out(c, b):
        return pltpu.make_async_copy(
            w1ob.at[b], w1o_hbm.at[pl.ds(c * _B1, _B1), :], sf1o.at[b])

    def f2_in(c, b):
        return pltpu.make_async_copy(
            f2_hbm.at[pl.ds(c * _B2, _B2), :], f2b.at[b], sf2i.at[b])

    def w2_out(c, b):
        return pltpu.make_async_copy(
            w2ob.at[b], w2o_hbm.at[pl.ds(c * _B2, _B2), :], sf2o.at[b])

    def r_in(c, b):
        return pltpu.make_async_copy(
            x_hbm.at[pl.ds(c * _RB, _RB), :], rib.at[b], sri.at[b])

    def r_out(c, b):
        return pltpu.make_async_copy(
            rob.at[b], y_hbm.at[pl.ds(c * _RB, _RB), :], sro.at[b])

    def relu_chunk(c):
        b = jax.lax.rem(c, _RNB)

        @pl.when(c >= _RNB)
        def _wait_out():
            r_out(c - _RNB, b).wait()

        r_in(c, b).wait()
        rob[b] = jnp.maximum(rib[b], 0.0)
        r_out(c, b).start()

        @pl.when(c + _RNB < _NR)
        def _next_in():
            r_in(c + _RNB, b).start()

    for i in range(_NBLK):
        w1_load(i).start()
    for i in range(_W2LC):
        w2_load(i).start()
    for b in range(_RNB):
        r_in(b, b).start()

    def mags1_step(i, _):
        w1_load(i).wait()
        w1 = w1v[pl.ds(i * _MB, _MB), :]
        s1 = jnp.sum(w1 * w1, axis=1)
        s1_s[pl.ds(i, 1), :] = jnp.sqrt(s1).reshape(1, _MB)
        return 0

    jax.lax.fori_loop(0, _NBLK, mags1_step, 0)

    for i in range(_W2LC):
        w2_load(i).wait()

    for b in range(_NB):
        f1_in(b, b).start()
        f2_in(b, b).start()

    def mags2_step(i, _):
        w2 = w2v[:, pl.ds(i * _MB, _MB)]
        s2 = jnp.sum(w2 * w2, axis=0)
        mags_s[pl.ds(i, 1), :] = (
            s1_s[pl.ds(i, 1), :] * jnp.sqrt(s2).reshape(1, _MB))
        return 0

    jax.lax.fori_loop(0, _NBLK, mags2_step, 0)

    mask = _bottom_k_mask(mags_s[...])
    mask_s[...] = mask
    pltpu.make_async_copy(mask_s, maskout_hbm, smask).start()
    for i in range(_NBLK):
        maskrow_s[0:1, pl.ds(i * _MB, _MB)] = mask_s[pl.ds(i, 1), :]

    def main_step(c, _):
        b = jax.lax.rem(c, _NB)

        @pl.when(c >= _NB)
        def _wait_out1():
            w1_out(c - _NB, b).wait()

        f1_in(c, b).wait()
        mrow = maskrow_s[0:1, pl.ds(c * _B1, _B1)]
        ii = jax.lax.broadcasted_iota(jnp.int32, (_B1, _B1), 0)
        jj = jax.lax.broadcasted_iota(jnp.int32, (_B1, _B1), 1)
        m_b = jnp.broadcast_to(mrow, (_B1, _B1))
        mcol = jnp.sum(jnp.where(ii == jj, m_b, 0.0), axis=1, keepdims=True)
        w1ob[b] = jnp.where(mcol > 0.5, w1v[pl.ds(c * _B1, _B1), :], f1b[b])
        w1_out(c, b).start()

        @pl.when(c + _NB < _N1)
        def _next_in1():
            f1_in(c + _NB, b).start()

        relu_chunk(4 * c)
        relu_chunk(4 * c + 1)

        @pl.when(c >= _NB)
        def _wait_out2():
            w2_out(c - _NB, b).wait()

        f2_in(c, b).wait()
        keep = maskrow_s[...] > 0.5
        w2ob[b] = jnp.where(keep, w2v[pl.ds(c * _B2, _B2), :], f2b[b])
        w2_out(c, b).start()

        @pl.when(c + _NB < _N2)
        def _next_in2():
            f2_in(c + _NB, b).start()

        relu_chunk(4 * c + 2)
        relu_chunk(4 * c + 3)
        return 0

    jax.lax.fori_loop(0, _N1, main_step, 0)

    for c in range(_N1 - _NB, _N1):
        w1_out(c, c % _NB).wait()
    for c in range(_N2 - _NB, _N2):
        w2_out(c, c % _NB).wait()
    for c in range(_NR - _RNB, _NR):
        r_out(c, c % _RNB).wait()
    pltpu.make_async_copy(mask_s, maskout_hbm, smask).wait()


def _fused_manual(W1, W2, frozen1, frozen2, x2):
    return pl.pallas_call(
        _fused_manual_body,
        in_specs=[pl.BlockSpec(memory_space=pl.ANY)] * 5,
        out_specs=[pl.BlockSpec(memory_space=pl.ANY)] * 4,
        out_shape=[
            jax.ShapeDtypeStruct((_NBLK, _MB), jnp.float32),
            jax.ShapeDtypeStruct((_DFF, _DMODEL), jnp.float32),
            jax.ShapeDtypeStruct((_DMODEL, _DFF), jnp.float32),
            jax.ShapeDtypeStruct((_XROWS, _DMODEL), jnp.float32),
        ],
        scratch_shapes=[
            pltpu.VMEM((_DFF, _DMODEL), jnp.float32),
            pltpu.VMEM((_DMODEL, _DFF), jnp.float32),
            pltpu.VMEM((_NBLK, _MB), jnp.float32),
            pltpu.VMEM((_NBLK, _MB), jnp.float32),
            pltpu.VMEM((_NBLK, _MB), jnp.float32),
            pltpu.VMEM((1, _DFF), jnp.float32),
            pltpu.VMEM((_NB, _B1, _DMODEL), jnp.float32),
            pltpu.VMEM((_NB, _B1, _DMODEL), jnp.float32),
            pltpu.VMEM((_NB, _B2, _DFF), jnp.float32),
            pltpu.VMEM((_NB, _B2, _DFF), jnp.float32),
            pltpu.VMEM((_RNB, _RB, _DMODEL), jnp.float32),
            pltpu.VMEM((_RNB, _RB, _DMODEL), jnp.float32),
            pltpu.SemaphoreType.DMA((_NBLK,)),
            pltpu.SemaphoreType.DMA((_W2LC,)),
            pltpu.SemaphoreType.DMA((_NB,)),
            pltpu.SemaphoreType.DMA((_NB,)),
            pltpu.SemaphoreType.DMA((_NB,)),
            pltpu.SemaphoreType.DMA((_NB,)),
            pltpu.SemaphoreType.DMA((_RNB,)),
            pltpu.SemaphoreType.DMA((_RNB,)),
            pltpu.SemaphoreType.DMA,
        ],
    )(W1, W2, frozen1, frozen2, x2)


def kernel(x, W1, W2, frozen1, frozen2, target1, target2):
    del target1, target2

    x2 = x.reshape(_XROWS, _DMODEL)

    mask2d, W1_new, W2_new, y = _fused_manual(W1, W2, frozen1, frozen2, x2)

    mask = mask2d.reshape(_DFF)
    return y.reshape(x.shape), W1_new, W2_new, mask
